# baseline (speedup 1.0000x reference)
import functools

import jax
import jax.numpy as jnp
from jax import lax
from jax.experimental import pallas as pl
from jax.experimental.pallas import tpu as pltpu
from jax.experimental.pallas import tpu_sc as plsc

_V = 10000
_D = 10000
_B = 2048
_NC = 2
_NS = 16
_NW = _NC * _NS
_BPW = _B // _NW
_K = 8
_NCHUNK = _BPW // _K


@functools.partial(
    pl.kernel,
    out_type=jax.ShapeDtypeStruct((_B, _D), jnp.float32),
    mesh=plsc.VectorSubcoreMesh(core_axis_name="c", subcore_axis_name="s"),
    scratch_types=[
        pltpu.VMEM((_BPW,), jnp.int32),
        pltpu.VMEM((_K, _D), jnp.float32),
        pltpu.SemaphoreType.DMA,
    ],
    compiler_params=pltpu.CompilerParams(use_tc_tiling_on_sc=False),
)
def _gather_rows(sims_hbm, idx_hbm, out_hbm, idx_v, rows_v, sem):
    wid = lax.axis_index("s") * _NC + lax.axis_index("c")
    base = wid * _BPW
    pltpu.sync_copy(idx_hbm.at[pl.ds(base, _BPW)], idx_v)
    for c in range(_NCHUNK):
        pltpu.async_copy(
            sims_hbm.at[idx_v.at[pl.ds(c * _K, _K)]], rows_v, sem
        ).wait()
        pltpu.sync_copy(rows_v, out_hbm.at[pl.ds(base + c * _K, _K)])


def kernel(x, sims):
    idx = x.reshape(-1).astype(jnp.int32)
    out = _gather_rows(sims, idx)
    return out.reshape(x.shape[0], x.shape[1], _V)


# speedup vs baseline: 1.0003x; 1.0003x over previous
import functools

import jax
import jax.numpy as jnp
from jax import lax
from jax.experimental import pallas as pl
from jax.experimental.pallas import tpu as pltpu
from jax.experimental.pallas import tpu_sc as plsc

_V = 10000
_D = 10000
_B = 2048
_NC = 2
_NS = 16
_NW = _NC * _NS
_BPW = _B // _NW
_K = 4
_NBUF = 3
_NCHUNK = _BPW // _K


@functools.partial(
    pl.kernel,
    out_type=jax.ShapeDtypeStruct((_B, _D), jnp.float32),
    mesh=plsc.VectorSubcoreMesh(core_axis_name="c", subcore_axis_name="s"),
    scratch_types=[
        pltpu.VMEM((_NCHUNK, _K), jnp.int32),
        [pltpu.VMEM((_K, _D), jnp.float32) for _ in range(_NBUF)],
        [pltpu.SemaphoreType.DMA for _ in range(_NBUF)],
        [pltpu.SemaphoreType.DMA for _ in range(_NBUF)],
    ],
    compiler_params=pltpu.CompilerParams(use_tc_tiling_on_sc=False),
)
def _gather_rows(sims_hbm, idx_hbm, out_hbm, idx_v, bufs, gsems, osems):
    wid = lax.axis_index("s") * _NC + lax.axis_index("c")
    base = wid * _BPW
    pltpu.sync_copy(idx_hbm.at[pl.ds(wid * _NCHUNK, _NCHUNK)], idx_v)

    def start_gather(c):
        b = c % _NBUF
        return pltpu.async_copy(
            sims_hbm.at[idx_v.at[c]], bufs[b], gsems[b]
        )

    gh = [None] * _NCHUNK
    oh = [None] * _NCHUNK
    for c in range(_NBUF):
        gh[c] = start_gather(c)
    for c in range(_NCHUNK):
        b = c % _NBUF
        gh[c].wait()
        oh[c] = pltpu.async_copy(
            bufs[b], out_hbm.at[pl.ds(base + c * _K, _K)], osems[b]
        )
        if c + _NBUF < _NCHUNK:
            oh[c].wait()
            gh[c + _NBUF] = start_gather(c + _NBUF)
    for c in range(_NCHUNK - _NBUF, _NCHUNK):
        oh[c].wait()


def kernel(x, sims):
    idx = x.reshape(_B // _K, _K).astype(jnp.int32)
    out = _gather_rows(sims, idx)
    return out.reshape(x.shape[0], x.shape[1], _V)


# speedup vs baseline: 5.9998x; 5.9979x over previous
import functools

import jax
import jax.numpy as jnp
from jax import lax
from jax.experimental import pallas as pl
from jax.experimental.pallas import tpu as pltpu
from jax.experimental.pallas import tpu_sc as plsc

_V = 10000
_D = 10000
_DM = 9984
_TW = 128
_TR = _D - _DM
_B = 2048
_NC = 2
_NS = 16
_NW = _NC * _NS
_BPW = _B // _NW
_K = 8
_NCHUNK = _BPW // _K


@functools.partial(
    pl.kernel,
    out_type=jax.ShapeDtypeStruct((_B, _D), jnp.float32),
    mesh=plsc.VectorSubcoreMesh(core_axis_name="c", subcore_axis_name="s"),
    scratch_types=[
        pltpu.VMEM((_BPW,), jnp.int32),
        pltpu.VMEM((_K, _D), jnp.float32),
        pltpu.VMEM((_K, _TW), jnp.float32),
        pltpu.SemaphoreType.DMA,
        pltpu.SemaphoreType.DMA,
    ],
)
def _gather_rows(sims_hbm, tail_hbm, idx_hbm, out_hbm, idx_v, rows_v, tail_v,
                 gsem, tsem):
    wid = lax.axis_index("s") * _NC + lax.axis_index("c")
    base = wid * _BPW
    pltpu.sync_copy(idx_hbm.at[pl.ds(base, _BPW)], idx_v)
    for c in range(_NCHUNK):
        idx_c = idx_v.at[pl.ds(c * _K, _K)]
        gh = pltpu.async_copy(
            sims_hbm.at[idx_c, pl.ds(0, _DM)], rows_v.at[:, pl.ds(0, _DM)],
            gsem,
        )
        th = pltpu.async_copy(tail_hbm.at[idx_c], tail_v, tsem)
        gh.wait()
        th.wait()
        for r in range(_K):
            rows_v[r, pl.ds(_DM, _TR)] = tail_v[r, pl.ds(_TW - _TR, _TR)]
        pltpu.sync_copy(rows_v, out_hbm.at[pl.ds(base + c * _K, _K)])


def kernel(x, sims):
    idx = x.reshape(-1).astype(jnp.int32)
    tail = lax.slice(sims, (0, _D - _TW), (_V, _D))
    out = _gather_rows(sims, tail, idx)
    return out.reshape(x.shape[0], x.shape[1], _V)
